# baseline (speedup 1.0000x reference)
import functools

import numpy as np
import jax
import jax.numpy as jnp
from jax import lax
from jax.experimental import pallas as pl
from jax.experimental.pallas import tpu as pltpu
from jax.experimental.pallas import tpu_sc as plsc

_BSZ = 4096
_HYPO = 767
_DIM = 10
_SPAN = 1023

_ROT0 = (13, 15, 26, 6)
_ROT1 = (17, 29, 16, 24)


def _np_threefry2x32(k1, k2, x0, x1):
    k1 = np.uint32(k1)
    k2 = np.uint32(k2)
    x0 = np.asarray(x0, np.uint32).copy()
    x1 = np.asarray(x1, np.uint32).copy()
    ks = [k1, k2, np.uint32(k1 ^ k2 ^ np.uint32(0x1BD11BDA))]

    def rotl(v, r):
        return np.uint32((v << np.uint32(r)) | (v >> np.uint32(32 - r)))

    x0 += ks[0]
    x1 += ks[1]
    inject = [(1, 2, 1), (2, 0, 2), (0, 1, 3), (1, 2, 4), (2, 0, 5)]
    rots = [_ROT0, _ROT1, _ROT0, _ROT1, _ROT0]
    for (ia, ib, c), rr in zip(inject, rots):
        for r in rr:
            x0 = np.uint32(x0 + x1)
            x1 = rotl(x1, r)
            x1 = np.uint32(x0 ^ x1)
        x0 = np.uint32(x0 + ks[ia])
        x1 = np.uint32(x1 + ks[ib] + np.uint32(c))
    return x0, x1


_B1, _B2 = _np_threefry2x32(0, 42, np.array([0, 0]), np.array([0, 1]))
_K1A, _K1B = int(_B1[0]), int(_B2[0])
_K2A, _K2B = int(_B1[1]), int(_B2[1])


def _tf_bits(k1, k2, x1):
    ks0 = jnp.uint32(k1)
    ks1 = jnp.uint32(k2)
    ks2 = jnp.uint32(k1 ^ k2 ^ 0x1BD11BDA)
    inject = [(ks1, ks2, 1), (ks2, ks0, 2), (ks0, ks1, 3),
              (ks1, ks2, 4), (ks2, ks0, 5)]
    rots = [_ROT0, _ROT1, _ROT0, _ROT1, _ROT0]
    x0 = None
    for (ka, kb, c), rr in zip(inject, rots):
        for r in rr:
            x0 = (x1 + ks0) if x0 is None else (x0 + x1)
            x1 = (x1 << jnp.uint32(r)) | (x1 >> jnp.uint32(32 - r))
            x1 = x0 ^ x1
        x0 = x0 + ka
        x1 = x1 + (kb + jnp.uint32(c))
    return x0 ^ x1


def _modsum1023(x):
    m = jnp.uint32(_SPAN)
    s = (x & m) + ((x >> jnp.uint32(10)) & m) + \
        ((x >> jnp.uint32(20)) & m) + (x >> jnp.uint32(30))
    return (s & m) + (s >> jnp.uint32(10))


def _body(x_ref, pa_ref, pb_ref, out_ref, idx_ref):
    rows = idx_ref.shape[0]
    base = jnp.uint32(rows * _HYPO) * pl.program_id(0).astype(jnp.uint32)

    hi = _tf_bits(_K1A, _K1B, pa_ref[...] + base)
    lo = _tf_bits(_K2A, _K2B, pb_ref[...] + base)
    acc = (_modsum1023(hi) << jnp.uint32(2)) + _modsum1023(lo)
    s = (acc & jnp.uint32(_SPAN)) + (acc >> jnp.uint32(10))
    off = jnp.where(s >= jnp.uint32(_SPAN), s - jnp.uint32(_SPAN), s)
    idx_ref[...] = off.view(jnp.int32)

    g = off.view(jnp.int32) + 1
    for d in range(_DIM):
        mask = (g << (22 + d)) >> 31
        xd = x_ref[:, d].reshape(rows, 1).view(jnp.int32)
        out_ref[d] = (mask & xd).view(jnp.float32)


def _sc_rows(x_pad, start_row, nrows_total):
    n_workers = 32
    per_w = nrows_total // n_workers
    groups = per_w // 8
    mesh = plsc.VectorSubcoreMesh(core_axis_name="c", subcore_axis_name="s")

    @functools.partial(
        pl.kernel,
        out_type=[
            jax.ShapeDtypeStruct((_DIM, nrows_total, 768), jnp.float32),
            jax.ShapeDtypeStruct((nrows_total, 768), jnp.int32),
        ],
        scratch_types=[
            pltpu.VMEM((8, 16), jnp.float32),
            pltpu.VMEM((_DIM, 8, 768), jnp.float32),
            pltpu.VMEM((8, 768), jnp.int32),
        ],
        mesh=mesh,
    )
    def body(x_ref, out_ref, idx_ref, xrows_v, planes_v, idxrows_v):
        wid = lax.axis_index("s") * 2 + lax.axis_index("c")

        def group_body(gi, carry):
            b0 = wid * per_w + gi * 8
            pltpu.sync_copy(x_ref.at[pl.ds(start_row + b0, 8)], xrows_v)
            for r in range(8):
                xrow = xrows_v[r, pl.ds(0, 16)]
                xd = [
                    lax.bitcast_convert_type(
                        jnp.full((16,), xrow[d], jnp.float32), jnp.int32)
                    for d in range(_DIM)
                ]
                row_p = ((start_row + b0 + r) * _HYPO).astype(jnp.uint32)

                def chunk_body(c, carry2, _r=r, _xd=xd, _row_p=row_p):
                    hbase = c * 16
                    lanes = lax.iota(jnp.uint32, 16) + \
                        (_row_p + hbase.astype(jnp.uint32))
                    hi = _tf_bits(_K1A, _K1B, lanes + jnp.uint32(_K1B))
                    lo = _tf_bits(_K2A, _K2B, lanes + jnp.uint32(_K2B))
                    acc = (_modsum1023(hi) << jnp.uint32(2)) + _modsum1023(lo)
                    s = (acc & jnp.uint32(_SPAN)) + (acc >> jnp.uint32(10))
                    off = jnp.where(s >= jnp.uint32(_SPAN),
                                    s - jnp.uint32(_SPAN), s)
                    off = lax.bitcast_convert_type(off, jnp.int32)
                    idxrows_v[_r, pl.ds(hbase, 16)] = off
                    g = off + 1
                    for d in range(_DIM):
                        mask = (g << (22 + d)) >> 31
                        planes_v[d, _r, pl.ds(hbase, 16)] = \
                            lax.bitcast_convert_type(mask & _xd[d], jnp.float32)
                    return carry2

                lax.fori_loop(0, 48, chunk_body, 0, unroll=False)
            pltpu.sync_copy(idxrows_v, idx_ref.at[pl.ds(b0, 8)])
            for d in range(_DIM):
                pltpu.sync_copy(planes_v.at[d], out_ref.at[d, pl.ds(b0, 8)])
            return carry

        lax.fori_loop(0, groups, group_body, 0, unroll=False)

    return body(x_pad)


def _kernel_sc_full(x):
    x_pad = jnp.pad(x, ((0, 0), (0, 6)))
    out3, idx = _sc_rows(x_pad, 0, _BSZ)
    return jnp.transpose(out3[:, :, :_HYPO], (1, 2, 0)), idx[:, :_HYPO]


def kernel(x):
    return _kernel_sc_full(x)


def _kernel_tc(x):
    rows = 32
    grid = _BSZ // rows
    p0 = (np.arange(rows, dtype=np.uint32)[:, None] * np.uint32(_HYPO)
          + np.arange(_HYPO, dtype=np.uint32)[None, :])
    pa = jnp.asarray(p0 + np.uint32(_K1B))
    pb = jnp.asarray(p0 + np.uint32(_K2B))
    out3, idx = pl.pallas_call(
        _body,
        grid=(grid,),
        in_specs=[
            pl.BlockSpec((rows, _DIM), lambda i: (i, 0)),
            pl.BlockSpec((rows, _HYPO), lambda i: (0, 0)),
            pl.BlockSpec((rows, _HYPO), lambda i: (0, 0)),
        ],
        out_specs=[
            pl.BlockSpec((_DIM, rows, _HYPO), lambda i: (0, i, 0)),
            pl.BlockSpec((rows, _HYPO), lambda i: (i, 0)),
        ],
        out_shape=[
            jax.ShapeDtypeStruct((_DIM, _BSZ, _HYPO), jnp.float32),
            jax.ShapeDtypeStruct((_BSZ, _HYPO), jnp.int32),
        ],
    )(x, pa, pb)
    return jnp.transpose(out3, (1, 2, 0)), idx


# speedup vs baseline: 3.3493x; 3.3493x over previous
import functools

import numpy as np
import jax
import jax.numpy as jnp
from jax import lax
from jax.experimental import pallas as pl
from jax.experimental.pallas import tpu as pltpu
from jax.experimental.pallas import tpu_sc as plsc

_BSZ = 4096
_HYPO = 767
_DIM = 10
_SPAN = 1023

_ROT0 = (13, 15, 26, 6)
_ROT1 = (17, 29, 16, 24)


def _np_threefry2x32(k1, k2, x0, x1):
    k1 = np.uint32(k1)
    k2 = np.uint32(k2)
    x0 = np.asarray(x0, np.uint32).copy()
    x1 = np.asarray(x1, np.uint32).copy()
    ks = [k1, k2, np.uint32(k1 ^ k2 ^ np.uint32(0x1BD11BDA))]

    def rotl(v, r):
        return np.uint32((v << np.uint32(r)) | (v >> np.uint32(32 - r)))

    x0 += ks[0]
    x1 += ks[1]
    inject = [(1, 2, 1), (2, 0, 2), (0, 1, 3), (1, 2, 4), (2, 0, 5)]
    rots = [_ROT0, _ROT1, _ROT0, _ROT1, _ROT0]
    for (ia, ib, c), rr in zip(inject, rots):
        for r in rr:
            x0 = np.uint32(x0 + x1)
            x1 = rotl(x1, r)
            x1 = np.uint32(x0 ^ x1)
        x0 = np.uint32(x0 + ks[ia])
        x1 = np.uint32(x1 + ks[ib] + np.uint32(c))
    return x0, x1


_B1, _B2 = _np_threefry2x32(0, 42, np.array([0, 0]), np.array([0, 1]))
_K1A, _K1B = int(_B1[0]), int(_B2[0])
_K2A, _K2B = int(_B1[1]), int(_B2[1])


def _tf_bits(k1, k2, x1):
    ks0 = jnp.uint32(k1)
    ks1 = jnp.uint32(k2)
    ks2 = jnp.uint32(k1 ^ k2 ^ 0x1BD11BDA)
    inject = [(ks1, ks2, 1), (ks2, ks0, 2), (ks0, ks1, 3),
              (ks1, ks2, 4), (ks2, ks0, 5)]
    rots = [_ROT0, _ROT1, _ROT0, _ROT1, _ROT0]
    x0 = None
    for (ka, kb, c), rr in zip(inject, rots):
        for r in rr:
            x0 = (x1 + ks0) if x0 is None else (x0 + x1)
            x1 = (x1 << jnp.uint32(r)) | (x1 >> jnp.uint32(32 - r))
            x1 = x0 ^ x1
        x0 = x0 + ka
        x1 = x1 + (kb + jnp.uint32(c))
    return x0 ^ x1


def _modsum1023(x):
    m = jnp.uint32(_SPAN)
    s = (x & m) + ((x >> jnp.uint32(10)) & m) + \
        ((x >> jnp.uint32(20)) & m) + (x >> jnp.uint32(30))
    return (s & m) + (s >> jnp.uint32(10))


def _body(x_ref, pa_ref, pb_ref, out_ref, idx_ref):
    rows = idx_ref.shape[0]
    base = jnp.uint32(rows * _HYPO) * pl.program_id(0).astype(jnp.uint32)

    hi = _tf_bits(_K1A, _K1B, pa_ref[...] + base)
    lo = _tf_bits(_K2A, _K2B, pb_ref[...] + base)
    acc = (_modsum1023(hi) << jnp.uint32(2)) + _modsum1023(lo)
    s = (acc & jnp.uint32(_SPAN)) + (acc >> jnp.uint32(10))
    off = jnp.where(s >= jnp.uint32(_SPAN), s - jnp.uint32(_SPAN), s)
    idx_ref[...] = off.view(jnp.int32)

    g = off.view(jnp.int32) + 1
    for d in range(_DIM):
        mask = (g << (22 + d)) >> 31
        xd = x_ref[:, d].reshape(rows, 1).view(jnp.int32)
        out_ref[d] = (mask & xd).view(jnp.float32)


def _sc_rows(x_pad, start_row, nrows_total):
    n_workers = 32
    per_w = nrows_total // n_workers
    groups = per_w // 8
    mesh = plsc.VectorSubcoreMesh(core_axis_name="c", subcore_axis_name="s")

    @functools.partial(
        pl.kernel,
        out_type=[
            jax.ShapeDtypeStruct((_DIM, nrows_total, 768), jnp.float32),
            jax.ShapeDtypeStruct((nrows_total, 768), jnp.int32),
        ],
        scratch_types=[
            pltpu.VMEM((8, 16), jnp.float32),
            pltpu.VMEM((_DIM, 8, 768), jnp.float32),
            pltpu.VMEM((8, 768), jnp.int32),
        ],
        mesh=mesh,
    )
    def body(x_ref, out_ref, idx_ref, xrows_v, planes_v, idxrows_v):
        wid = lax.axis_index("s") * 2 + lax.axis_index("c")

        def group_body(gi, carry):
            b0 = wid * per_w + gi * 8
            pltpu.sync_copy(x_ref.at[pl.ds(start_row + b0, 8)], xrows_v)
            for r in range(8):
                xrow = xrows_v[r, pl.ds(0, 16)]
                xd = [
                    lax.bitcast_convert_type(
                        jnp.full((16,), xrow[d], jnp.float32), jnp.int32)
                    for d in range(_DIM)
                ]
                row_p = ((start_row + b0 + r) * _HYPO).astype(jnp.uint32)

                def chunk_body(c, carry2, _r=r, _xd=xd, _row_p=row_p):
                    hbase = c * 16
                    lanes = lax.iota(jnp.uint32, 16) + \
                        (_row_p + hbase.astype(jnp.uint32))
                    hi = _tf_bits(_K1A, _K1B, lanes + jnp.uint32(_K1B))
                    lo = _tf_bits(_K2A, _K2B, lanes + jnp.uint32(_K2B))
                    acc = (_modsum1023(hi) << jnp.uint32(2)) + _modsum1023(lo)
                    s = (acc & jnp.uint32(_SPAN)) + (acc >> jnp.uint32(10))
                    off = jnp.where(s >= jnp.uint32(_SPAN),
                                    s - jnp.uint32(_SPAN), s)
                    off = lax.bitcast_convert_type(off, jnp.int32)
                    idxrows_v[_r, pl.ds(hbase, 16)] = off
                    g = off + 1
                    for d in range(_DIM):
                        mask = (g << (22 + d)) >> 31
                        planes_v[d, _r, pl.ds(hbase, 16)] = \
                            lax.bitcast_convert_type(mask & _xd[d], jnp.float32)
                    return carry2

                lax.fori_loop(0, 48, chunk_body, 0, unroll=False)
            pltpu.sync_copy(idxrows_v, idx_ref.at[pl.ds(b0, 8)])
            for d in range(_DIM):
                pltpu.sync_copy(planes_v.at[d], out_ref.at[d, pl.ds(b0, 8)])
            return carry

        lax.fori_loop(0, groups, group_body, 0, unroll=False)

    return body(x_pad)


def _kernel_sc_full(x):
    x_pad = jnp.pad(x, ((0, 0), (0, 6)))
    out3, idx = _sc_rows(x_pad, 0, _BSZ)
    return jnp.transpose(out3[:, :, :_HYPO], (1, 2, 0)), idx[:, :_HYPO]


def kernel(x):
    return _kernel_hybrid(x)


def _kernel_hybrid(x, tc_rows=3072):
    sc_rows = _BSZ - tc_rows
    x_pad = jnp.pad(x, ((0, 0), (0, 6)))
    sc_out, sc_idx = _sc_rows(x_pad, tc_rows, sc_rows)
    out3, idx = _tc_pallas(x, nrows=tc_rows)
    out3 = lax.dynamic_update_slice(out3, sc_out[:, :, :_HYPO],
                                    (0, tc_rows, 0))
    idx = lax.dynamic_update_slice(idx, sc_idx[:, :_HYPO], (tc_rows, 0))
    return jnp.transpose(out3, (1, 2, 0)), idx


def _kernel_tc(x):
    out3, idx = _tc_pallas(x, nrows=_BSZ)
    return jnp.transpose(out3, (1, 2, 0)), idx


def _tc_pallas(x, nrows):
    rows = 32
    grid = nrows // rows
    p0 = (np.arange(rows, dtype=np.uint32)[:, None] * np.uint32(_HYPO)
          + np.arange(_HYPO, dtype=np.uint32)[None, :])
    pa = jnp.asarray(p0 + np.uint32(_K1B))
    pb = jnp.asarray(p0 + np.uint32(_K2B))
    out3, idx = pl.pallas_call(
        _body,
        grid=(grid,),
        in_specs=[
            pl.BlockSpec((rows, _DIM), lambda i: (i, 0)),
            pl.BlockSpec((rows, _HYPO), lambda i: (0, 0)),
            pl.BlockSpec((rows, _HYPO), lambda i: (0, 0)),
        ],
        out_specs=[
            pl.BlockSpec((_DIM, rows, _HYPO), lambda i: (0, i, 0)),
            pl.BlockSpec((rows, _HYPO), lambda i: (i, 0)),
        ],
        out_shape=[
            jax.ShapeDtypeStruct((_DIM, _BSZ, _HYPO), jnp.float32),
            jax.ShapeDtypeStruct((_BSZ, _HYPO), jnp.int32),
        ],
    )(x, pa, pb)
    return out3, idx


# speedup vs baseline: 3.5218x; 1.0515x over previous
import functools

import numpy as np
import jax
import jax.numpy as jnp
from jax import lax
from jax.experimental import pallas as pl
from jax.experimental.pallas import tpu as pltpu
from jax.experimental.pallas import tpu_sc as plsc

_BSZ = 4096
_HYPO = 767
_DIM = 10
_SPAN = 1023

_ROT0 = (13, 15, 26, 6)
_ROT1 = (17, 29, 16, 24)


def _np_threefry2x32(k1, k2, x0, x1):
    k1 = np.uint32(k1)
    k2 = np.uint32(k2)
    x0 = np.asarray(x0, np.uint32).copy()
    x1 = np.asarray(x1, np.uint32).copy()
    ks = [k1, k2, np.uint32(k1 ^ k2 ^ np.uint32(0x1BD11BDA))]

    def rotl(v, r):
        return np.uint32((v << np.uint32(r)) | (v >> np.uint32(32 - r)))

    x0 += ks[0]
    x1 += ks[1]
    inject = [(1, 2, 1), (2, 0, 2), (0, 1, 3), (1, 2, 4), (2, 0, 5)]
    rots = [_ROT0, _ROT1, _ROT0, _ROT1, _ROT0]
    for (ia, ib, c), rr in zip(inject, rots):
        for r in rr:
            x0 = np.uint32(x0 + x1)
            x1 = rotl(x1, r)
            x1 = np.uint32(x0 ^ x1)
        x0 = np.uint32(x0 + ks[ia])
        x1 = np.uint32(x1 + ks[ib] + np.uint32(c))
    return x0, x1


_B1, _B2 = _np_threefry2x32(0, 42, np.array([0, 0]), np.array([0, 1]))
_K1A, _K1B = int(_B1[0]), int(_B2[0])
_K2A, _K2B = int(_B1[1]), int(_B2[1])


def _tf_bits(k1, k2, x1):
    ks0 = jnp.uint32(k1)
    ks1 = jnp.uint32(k2)
    ks2 = jnp.uint32(k1 ^ k2 ^ 0x1BD11BDA)
    inject = [(ks1, ks2, 1), (ks2, ks0, 2), (ks0, ks1, 3),
              (ks1, ks2, 4), (ks2, ks0, 5)]
    rots = [_ROT0, _ROT1, _ROT0, _ROT1, _ROT0]
    x0 = None
    for (ka, kb, c), rr in zip(inject, rots):
        for r in rr:
            x0 = (x1 + ks0) if x0 is None else (x0 + x1)
            x1 = (x1 << jnp.uint32(r)) | (x1 >> jnp.uint32(32 - r))
            x1 = x0 ^ x1
        x0 = x0 + ka
        x1 = x1 + (kb + jnp.uint32(c))
    return x0 ^ x1


def _modsum1023(x):
    m = jnp.uint32(_SPAN)
    s = (x & m) + ((x >> jnp.uint32(10)) & m) + (x >> jnp.uint32(20))
    return (s & m) + (s >> jnp.uint32(10))


def _body(x_ref, pa_ref, pb_ref, out_ref, idx_ref):
    rows = idx_ref.shape[0]

    hi = _tf_bits(_K1A, _K1B, pa_ref[...])
    lo = _tf_bits(_K2A, _K2B, pb_ref[...])
    acc = (_modsum1023(hi) << jnp.uint32(2)) + _modsum1023(lo)
    s = (acc & jnp.uint32(_SPAN)) + (acc >> jnp.uint32(10))
    off = jnp.where(s >= jnp.uint32(_SPAN), s - jnp.uint32(_SPAN), s)
    idx_ref[...] = off.view(jnp.int32)

    g = off.view(jnp.int32) + 1
    for d in range(_DIM):
        mask = (g << (22 + d)) >> 31
        xd = x_ref[:, d].reshape(rows, 1).view(jnp.int32)
        out_ref[d] = (mask & xd).view(jnp.float32)


def _sc_rows(x_pad, start_row, nrows_total):
    n_workers = 32
    per_w = nrows_total // n_workers
    groups = per_w // 8
    mesh = plsc.VectorSubcoreMesh(core_axis_name="c", subcore_axis_name="s")

    @functools.partial(
        pl.kernel,
        out_type=[
            jax.ShapeDtypeStruct((_DIM, nrows_total, 768), jnp.float32),
            jax.ShapeDtypeStruct((nrows_total, 768), jnp.int32),
        ],
        scratch_types=[
            pltpu.VMEM((8, 16), jnp.float32),
            pltpu.VMEM((_DIM, 8, 768), jnp.float32),
            pltpu.VMEM((8, 768), jnp.int32),
        ],
        mesh=mesh,
    )
    def body(x_ref, out_ref, idx_ref, xrows_v, planes_v, idxrows_v):
        wid = lax.axis_index("s") * 2 + lax.axis_index("c")

        def group_body(gi, carry):
            b0 = wid * per_w + gi * 8
            pltpu.sync_copy(x_ref.at[pl.ds(start_row + b0, 8)], xrows_v)
            for r in range(8):
                xrow = xrows_v[r, pl.ds(0, 16)]
                xd = [
                    lax.bitcast_convert_type(
                        jnp.full((16,), xrow[d], jnp.float32), jnp.int32)
                    for d in range(_DIM)
                ]
                row_p = ((start_row + b0 + r) * _HYPO).astype(jnp.uint32)

                def chunk_body(c, carry2, _r=r, _xd=xd, _row_p=row_p):
                    hbase = c * 16
                    lanes = lax.iota(jnp.uint32, 16) + \
                        (_row_p + hbase.astype(jnp.uint32))
                    hi = _tf_bits(_K1A, _K1B, lanes + jnp.uint32(_K1B))
                    lo = _tf_bits(_K2A, _K2B, lanes + jnp.uint32(_K2B))
                    acc = (_modsum1023(hi) << jnp.uint32(2)) + _modsum1023(lo)
                    s = (acc & jnp.uint32(_SPAN)) + (acc >> jnp.uint32(10))
                    off = jnp.where(s >= jnp.uint32(_SPAN),
                                    s - jnp.uint32(_SPAN), s)
                    off = lax.bitcast_convert_type(off, jnp.int32)
                    idxrows_v[_r, pl.ds(hbase, 16)] = off
                    g = off + 1
                    for d in range(_DIM):
                        mask = (g << (22 + d)) >> 31
                        planes_v[d, _r, pl.ds(hbase, 16)] = \
                            lax.bitcast_convert_type(mask & _xd[d], jnp.float32)
                    return carry2

                lax.fori_loop(0, 48, chunk_body, 0, unroll=False)
            pltpu.sync_copy(idxrows_v, idx_ref.at[pl.ds(b0, 8)])
            for d in range(_DIM):
                pltpu.sync_copy(planes_v.at[d], out_ref.at[d, pl.ds(b0, 8)])
            return carry

        lax.fori_loop(0, groups, group_body, 0, unroll=False)

    return body(x_pad)


def _kernel_sc_full(x):
    x_pad = jnp.pad(x, ((0, 0), (0, 6)))
    out3, idx = _sc_rows(x_pad, 0, _BSZ)
    return jnp.transpose(out3[:, :, :_HYPO], (1, 2, 0)), idx[:, :_HYPO]


def kernel(x):
    return _kernel_tc(x)


def _kernel_hybrid(x, tc_rows=3072):
    sc_rows = _BSZ - tc_rows
    x_pad = jnp.pad(x, ((0, 0), (0, 6)))
    sc_out, sc_idx = _sc_rows(x_pad, tc_rows, sc_rows)
    out3, idx = _tc_pallas(x, nrows=tc_rows)
    out3 = lax.dynamic_update_slice(out3, sc_out[:, :, :_HYPO],
                                    (0, tc_rows, 0))
    idx = lax.dynamic_update_slice(idx, sc_idx[:, :_HYPO], (tc_rows, 0))
    return jnp.transpose(out3, (1, 2, 0)), idx


def _kernel_tc(x):
    out3, idx = _tc_pallas(x, nrows=_BSZ)
    return jnp.transpose(out3, (1, 2, 0)), idx


def _tc_pallas(x, nrows):
    rows = 32
    grid = nrows // rows
    p0 = (np.arange(_BSZ, dtype=np.uint32)[:, None] * np.uint32(_HYPO)
          + np.arange(_HYPO, dtype=np.uint32)[None, :])
    pa = jnp.asarray(p0 + np.uint32(_K1B))
    pb = jnp.asarray(p0 + np.uint32(_K2B))
    out3, idx = pl.pallas_call(
        _body,
        grid=(grid,),
        in_specs=[
            pl.BlockSpec((rows, _DIM), lambda i: (i, 0)),
            pl.BlockSpec((rows, _HYPO), lambda i: (i, 0)),
            pl.BlockSpec((rows, _HYPO), lambda i: (i, 0)),
        ],
        out_specs=[
            pl.BlockSpec((_DIM, rows, _HYPO), lambda i: (0, i, 0)),
            pl.BlockSpec((rows, _HYPO), lambda i: (i, 0)),
        ],
        out_shape=[
            jax.ShapeDtypeStruct((_DIM, _BSZ, _HYPO), jnp.float32),
            jax.ShapeDtypeStruct((_BSZ, _HYPO), jnp.int32),
        ],
    )(x, pa, pb)
    return out3, idx


# speedup vs baseline: 3.8343x; 1.0887x over previous
import functools

import numpy as np
import jax
import jax.numpy as jnp
from jax import lax
from jax.experimental import pallas as pl
from jax.experimental.pallas import tpu as pltpu
from jax.experimental.pallas import tpu_sc as plsc

_BSZ = 4096
_HYPO = 767
_DIM = 10
_SPAN = 1023

_ROT0 = (13, 15, 26, 6)
_ROT1 = (17, 29, 16, 24)


def _np_threefry2x32(k1, k2, x0, x1):
    k1 = np.uint32(k1)
    k2 = np.uint32(k2)
    x0 = np.asarray(x0, np.uint32).copy()
    x1 = np.asarray(x1, np.uint32).copy()
    ks = [k1, k2, np.uint32(k1 ^ k2 ^ np.uint32(0x1BD11BDA))]

    def rotl(v, r):
        return np.uint32((v << np.uint32(r)) | (v >> np.uint32(32 - r)))

    x0 += ks[0]
    x1 += ks[1]
    inject = [(1, 2, 1), (2, 0, 2), (0, 1, 3), (1, 2, 4), (2, 0, 5)]
    rots = [_ROT0, _ROT1, _ROT0, _ROT1, _ROT0]
    for (ia, ib, c), rr in zip(inject, rots):
        for r in rr:
            x0 = np.uint32(x0 + x1)
            x1 = rotl(x1, r)
            x1 = np.uint32(x0 ^ x1)
        x0 = np.uint32(x0 + ks[ia])
        x1 = np.uint32(x1 + ks[ib] + np.uint32(c))
    return x0, x1


_B1, _B2 = _np_threefry2x32(0, 42, np.array([0, 0]), np.array([0, 1]))
_K1A, _K1B = int(_B1[0]), int(_B2[0])
_K2A, _K2B = int(_B1[1]), int(_B2[1])


def _tf_bits(k1, k2, x1):
    ks0 = jnp.uint32(k1)
    ks1 = jnp.uint32(k2)
    ks2 = jnp.uint32(k1 ^ k2 ^ 0x1BD11BDA)
    inject = [(ks1, ks2, 1), (ks2, ks0, 2), (ks0, ks1, 3),
              (ks1, ks2, 4), (ks2, ks0, 5)]
    rots = [_ROT0, _ROT1, _ROT0, _ROT1, _ROT0]
    x0 = None
    for (ka, kb, c), rr in zip(inject, rots):
        for r in rr:
            x0 = (x1 + ks0) if x0 is None else (x0 + x1)
            x1 = (x1 << jnp.uint32(r)) | (x1 >> jnp.uint32(32 - r))
            x1 = x0 ^ x1
        x0 = x0 + ka
        x1 = x1 + (kb + jnp.uint32(c))
    return x0 ^ x1


def _modsum1023(x):
    m = jnp.uint32(_SPAN)
    s = (x & m) + ((x >> jnp.uint32(10)) & m) + (x >> jnp.uint32(20))
    return (s & m) + (s >> jnp.uint32(10))


def _body(x_ref, pa_ref, pb_ref, out_ref, idx_ref):
    rows = idx_ref.shape[0]

    base = jnp.uint32(rows * _HYPO) * pl.program_id(0).astype(jnp.uint32)
    hi = _tf_bits(_K1A, _K1B, pa_ref[...] + base)
    lo = _tf_bits(_K2A, _K2B, pb_ref[...] + base)
    acc = (_modsum1023(hi) << jnp.uint32(2)) + _modsum1023(lo)
    s = (acc & jnp.uint32(_SPAN)) + (acc >> jnp.uint32(10))
    off = jnp.where(s >= jnp.uint32(_SPAN), s - jnp.uint32(_SPAN), s)
    idx_ref[...] = off.view(jnp.int32)

    g = off.view(jnp.int32) + 1
    for d in range(_DIM):
        mask = (g << (22 + d)) >> 31
        xd = x_ref[:, d].reshape(rows, 1).view(jnp.int32)
        out_ref[d] = (mask & xd).view(jnp.float32)


def _sc_rows(x_pad, start_row, nrows_total):
    n_workers = 32
    per_w = nrows_total // n_workers
    groups = per_w // 8
    mesh = plsc.VectorSubcoreMesh(core_axis_name="c", subcore_axis_name="s")

    @functools.partial(
        pl.kernel,
        out_type=[
            jax.ShapeDtypeStruct((_DIM, nrows_total, 768), jnp.float32),
            jax.ShapeDtypeStruct((nrows_total, 768), jnp.int32),
        ],
        scratch_types=[
            pltpu.VMEM((8, 16), jnp.float32),
            pltpu.VMEM((_DIM, 8, 768), jnp.float32),
            pltpu.VMEM((8, 768), jnp.int32),
        ],
        mesh=mesh,
    )
    def body(x_ref, out_ref, idx_ref, xrows_v, planes_v, idxrows_v):
        wid = lax.axis_index("s") * 2 + lax.axis_index("c")

        def group_body(gi, carry):
            b0 = wid * per_w + gi * 8
            pltpu.sync_copy(x_ref.at[pl.ds(start_row + b0, 8)], xrows_v)
            for r in range(8):
                xrow = xrows_v[r, pl.ds(0, 16)]
                xd = [
                    lax.bitcast_convert_type(
                        jnp.full((16,), xrow[d], jnp.float32), jnp.int32)
                    for d in range(_DIM)
                ]
                row_p = ((start_row + b0 + r) * _HYPO).astype(jnp.uint32)

                def chunk_body(c, carry2, _r=r, _xd=xd, _row_p=row_p):
                    hbase = c * 16
                    lanes = lax.iota(jnp.uint32, 16) + \
                        (_row_p + hbase.astype(jnp.uint32))
                    hi = _tf_bits(_K1A, _K1B, lanes + jnp.uint32(_K1B))
                    lo = _tf_bits(_K2A, _K2B, lanes + jnp.uint32(_K2B))
                    acc = (_modsum1023(hi) << jnp.uint32(2)) + _modsum1023(lo)
                    s = (acc & jnp.uint32(_SPAN)) + (acc >> jnp.uint32(10))
                    off = jnp.where(s >= jnp.uint32(_SPAN),
                                    s - jnp.uint32(_SPAN), s)
                    off = lax.bitcast_convert_type(off, jnp.int32)
                    idxrows_v[_r, pl.ds(hbase, 16)] = off
                    g = off + 1
                    for d in range(_DIM):
                        mask = (g << (22 + d)) >> 31
                        planes_v[d, _r, pl.ds(hbase, 16)] = \
                            lax.bitcast_convert_type(mask & _xd[d], jnp.float32)
                    return carry2

                lax.fori_loop(0, 48, chunk_body, 0, unroll=False)
            pltpu.sync_copy(idxrows_v, idx_ref.at[pl.ds(b0, 8)])
            for d in range(_DIM):
                pltpu.sync_copy(planes_v.at[d], out_ref.at[d, pl.ds(b0, 8)])
            return carry

        lax.fori_loop(0, groups, group_body, 0, unroll=False)

    return body(x_pad)


def _kernel_sc_full(x):
    x_pad = jnp.pad(x, ((0, 0), (0, 6)))
    out3, idx = _sc_rows(x_pad, 0, _BSZ)
    return jnp.transpose(out3[:, :, :_HYPO], (1, 2, 0)), idx[:, :_HYPO]


def kernel(x):
    return _kernel_tc(x)


def _kernel_hybrid(x, tc_rows=3072):
    sc_rows = _BSZ - tc_rows
    x_pad = jnp.pad(x, ((0, 0), (0, 6)))
    sc_out, sc_idx = _sc_rows(x_pad, tc_rows, sc_rows)
    out3, idx = _tc_pallas(x, nrows=tc_rows)
    out3 = lax.dynamic_update_slice(out3, sc_out[:, :, :_HYPO],
                                    (0, tc_rows, 0))
    idx = lax.dynamic_update_slice(idx, sc_idx[:, :_HYPO], (tc_rows, 0))
    return jnp.transpose(out3, (1, 2, 0)), idx


def _kernel_tc(x):
    out3, idx = _tc_pallas(x, nrows=_BSZ)
    return jnp.transpose(out3, (1, 2, 0)), idx


def _tc_pallas(x, nrows):
    rows = 32
    grid = nrows // rows
    p0 = (np.arange(rows, dtype=np.uint32)[:, None] * np.uint32(_HYPO)
          + np.arange(_HYPO, dtype=np.uint32)[None, :])
    pa = jnp.asarray(p0 + np.uint32(_K1B))
    pb = jnp.asarray(p0 + np.uint32(_K2B))
    out3, idx = pl.pallas_call(
        _body,
        grid=(grid,),
        in_specs=[
            pl.BlockSpec((rows, _DIM), lambda i: (i, 0)),
            pl.BlockSpec((rows, _HYPO), lambda i: (0, 0)),
            pl.BlockSpec((rows, _HYPO), lambda i: (0, 0)),
        ],
        out_specs=[
            pl.BlockSpec((_DIM, rows, _HYPO), lambda i: (0, i, 0)),
            pl.BlockSpec((rows, _HYPO), lambda i: (i, 0)),
        ],
        out_shape=[
            jax.ShapeDtypeStruct((_DIM, _BSZ, _HYPO), jnp.float32),
            jax.ShapeDtypeStruct((_BSZ, _HYPO), jnp.int32),
        ],
    )(x, pa, pb)
    return out3, idx


# speedup vs baseline: 3.8424x; 1.0021x over previous
import numpy as np
import jax
import jax.numpy as jnp
from jax.experimental import pallas as pl

_BSZ = 4096
_HYPO = 767
_DIM = 10
_SPAN = 1023

_ROT0 = (13, 15, 26, 6)
_ROT1 = (17, 29, 16, 24)


def _np_threefry2x32(k1, k2, x0, x1):
    k1 = np.uint32(k1)
    k2 = np.uint32(k2)
    x0 = np.asarray(x0, np.uint32).copy()
    x1 = np.asarray(x1, np.uint32).copy()
    ks = [k1, k2, np.uint32(k1 ^ k2 ^ np.uint32(0x1BD11BDA))]

    def rotl(v, r):
        return np.uint32((v << np.uint32(r)) | (v >> np.uint32(32 - r)))

    x0 += ks[0]
    x1 += ks[1]
    inject = [(1, 2, 1), (2, 0, 2), (0, 1, 3), (1, 2, 4), (2, 0, 5)]
    rots = [_ROT0, _ROT1, _ROT0, _ROT1, _ROT0]
    for (ia, ib, c), rr in zip(inject, rots):
        for r in rr:
            x0 = np.uint32(x0 + x1)
            x1 = rotl(x1, r)
            x1 = np.uint32(x0 ^ x1)
        x0 = np.uint32(x0 + ks[ia])
        x1 = np.uint32(x1 + ks[ib] + np.uint32(c))
    return x0, x1


_B1, _B2 = _np_threefry2x32(0, 42, np.array([0, 0]), np.array([0, 1]))
_K1A, _K1B = int(_B1[0]), int(_B2[0])
_K2A, _K2B = int(_B1[1]), int(_B2[1])


def _tf_bits(k1, k2, x1):
    ks0 = jnp.uint32(k1)
    ks1 = jnp.uint32(k2)
    ks2 = jnp.uint32(k1 ^ k2 ^ 0x1BD11BDA)
    inject = [(ks1, ks2, 1), (ks2, ks0, 2), (ks0, ks1, 3),
              (ks1, ks2, 4), (ks2, ks0, 5)]
    rots = [_ROT0, _ROT1, _ROT0, _ROT1, _ROT0]
    x0 = None
    for (ka, kb, c), rr in zip(inject, rots):
        for r in rr:
            x0 = (x1 + ks0) if x0 is None else (x0 + x1)
            x1 = (x1 << jnp.uint32(r)) | (x1 >> jnp.uint32(32 - r))
            x1 = x0 ^ x1
        x0 = x0 + ka
        x1 = x1 + (kb + jnp.uint32(c))
    return x0 ^ x1


def _modsum1023(x):
    m = jnp.uint32(_SPAN)
    s = (x & m) + ((x >> jnp.uint32(10)) & m) + (x >> jnp.uint32(20))
    return (s & m) + (s >> jnp.uint32(10))


def _body(x_ref, pa_ref, pb_ref, out_ref, idx_ref):
    rows = idx_ref.shape[0]

    base = jnp.uint32(rows * _HYPO) * pl.program_id(0).astype(jnp.uint32)
    hi = _tf_bits(_K1A, _K1B, pa_ref[...] + base)
    lo = _tf_bits(_K2A, _K2B, pb_ref[...] + base)
    acc = (_modsum1023(hi) << jnp.uint32(2)) + _modsum1023(lo)
    s = (acc & jnp.uint32(_SPAN)) + (acc >> jnp.uint32(10))
    off = jnp.where(s >= jnp.uint32(_SPAN), s - jnp.uint32(_SPAN), s)
    idx_ref[...] = off.view(jnp.int32)

    g = off.view(jnp.int32) + 1
    for d in range(_DIM):
        mask = (g << (22 + d)) >> 31
        xd = x_ref[:, d].reshape(rows, 1).view(jnp.int32)
        out_ref[d] = (mask & xd).view(jnp.float32)


def kernel(x):
    rows = 32
    grid = _BSZ // rows
    p0 = (np.arange(rows, dtype=np.uint32)[:, None] * np.uint32(_HYPO)
          + np.arange(_HYPO, dtype=np.uint32)[None, :])
    pa = jnp.asarray(p0 + np.uint32(_K1B))
    pb = jnp.asarray(p0 + np.uint32(_K2B))
    out3, idx = pl.pallas_call(
        _body,
        grid=(grid,),
        in_specs=[
            pl.BlockSpec((rows, _DIM), lambda i: (i, 0)),
            pl.BlockSpec((rows, _HYPO), lambda i: (0, 0)),
            pl.BlockSpec((rows, _HYPO), lambda i: (0, 0)),
        ],
        out_specs=[
            pl.BlockSpec((_DIM, rows, _HYPO), lambda i: (0, i, 0)),
            pl.BlockSpec((rows, _HYPO), lambda i: (i, 0)),
        ],
        out_shape=[
            jax.ShapeDtypeStruct((_DIM, _BSZ, _HYPO), jnp.float32),
            jax.ShapeDtypeStruct((_BSZ, _HYPO), jnp.int32),
        ],
    )(x, pa, pb)
    return jnp.transpose(out3, (1, 2, 0)), idx
